# initial kernel scaffold (unmeasured)
import functools

import jax
import jax.numpy as jnp
from jax import lax
from jax.experimental import pallas as pl
from jax.experimental.pallas import tpu as pltpu


def kernel(x, dy):
    K, D = x.shape
    _, F = dy.shape
    G = F // 4
    H = D // 2
    CH = 512

    def body(x_hbm, dy_hbm, out_hbm, ld, xb, dyb, pk, zs, zr, xbuf, ybuf,
             ld_sem, st_sem, zs_sem, zr_sem, xs_sem, xr_sem, ys_sem, yr_sem):
        mx = lax.axis_index("x")
        my = lax.axis_index("y")
        mz = lax.axis_index("z")
        g = 2 * mx + my
        xpeer = (1 - mx, my, mz)
        ypeer = (mx, 1 - my, mz)
        zpeer = (mx, my, 1 - mz)

        bar = pltpu.get_barrier_semaphore()
        for dev in (xpeer, ypeer, zpeer):
            pl.semaphore_signal(bar, inc=1, device_id=dev,
                                device_id_type=pl.DeviceIdType.MESH)
        pl.semaphore_wait(bar, 3)

        def load_cast(src_hbm, col0, dst):
            for c in range(G // CH):
                cp = pltpu.make_async_copy(
                    src_hbm.at[:, pl.ds(col0 + c * CH, CH)], ld, ld_sem)
                cp.start()
                cp.wait()
                dst[:, c * CH:(c + 1) * CH] = ld[...].astype(jnp.bfloat16)

        load_cast(x_hbm, 0, xb)
        load_cast(dy_hbm, g * G, dyb)

        dn = (((0,), (0,)), ((), ()))

        zs[...] = lax.dot_general(
            xb[:, pl.ds((1 - mz) * H, H)], dyb[...], dn,
            preferred_element_type=jnp.float32).astype(jnp.bfloat16)
        zrdma = pltpu.make_async_remote_copy(
            zs, zr, zs_sem, zr_sem, device_id=zpeer,
            device_id_type=pl.DeviceIdType.MESH)
        zrdma.start()

        pk[...] = lax.dot_general(
            xb[:, pl.ds(mz * H, H)], dyb[...], dn,
            preferred_element_type=jnp.float32)

        zrdma.wait()
        pk[...] = pk[...] + zr[...].astype(jnp.float32)
        xbuf[0] = pk[...].astype(jnp.bfloat16)

        st = pltpu.make_async_copy(pk, out_hbm.at[:, pl.ds(g * G, G)], st_sem)
        st.start()

        xrdma = pltpu.make_async_remote_copy(
            xbuf.at[0], xbuf.at[1], xs_sem, xr_sem, device_id=xpeer,
            device_id_type=pl.DeviceIdType.MESH)
        xrdma.start()
        xrdma.wait()

        yrdma = pltpu.make_async_remote_copy(
            xbuf, ybuf, ys_sem, yr_sem, device_id=ypeer,
            device_id_type=pl.DeviceIdType.MESH)
        yrdma.start()

        st.wait()
        pk[...] = xbuf[1].astype(jnp.float32)
        st = pltpu.make_async_copy(
            pk, out_hbm.at[:, pl.ds((2 * (1 - mx) + my) * G, G)], st_sem)
        st.start()

        yrdma.wait()
        st.wait()
        pk[...] = ybuf[0].astype(jnp.float32)
        st = pltpu.make_async_copy(
            pk, out_hbm.at[:, pl.ds((2 * mx + (1 - my)) * G, G)], st_sem)
        st.start()
        st.wait()
        pk[...] = ybuf[1].astype(jnp.float32)
        st = pltpu.make_async_copy(
            pk, out_hbm.at[:, pl.ds((2 * (1 - mx) + (1 - my)) * G, G)], st_sem)
        st.start()
        st.wait()

        @functools.partial(pl.run_scoped, sem2=pltpu.SemaphoreType.REGULAR)
        def _(sem2):
            for dev in (xpeer, ypeer, zpeer):
                pl.semaphore_signal(sem2, inc=1, device_id=dev,
                                    device_id_type=pl.DeviceIdType.MESH)
            pl.semaphore_wait(sem2, 3)

    return pl.pallas_call(
        body,
        out_shape=jax.ShapeDtypeStruct((H, F), jnp.float32),
        in_specs=[
            pl.BlockSpec(memory_space=pltpu.ANY),
            pl.BlockSpec(memory_space=pltpu.ANY),
        ],
        out_specs=pl.BlockSpec(memory_space=pltpu.ANY),
        scratch_shapes=[
            pltpu.VMEM((K, CH), jnp.float32),
            pltpu.VMEM((K, D), jnp.bfloat16),
            pltpu.VMEM((K, G), jnp.bfloat16),
            pltpu.VMEM((H, G), jnp.float32),
            pltpu.VMEM((H, G), jnp.bfloat16),
            pltpu.VMEM((H, G), jnp.bfloat16),
            pltpu.VMEM((2, H, G), jnp.bfloat16),
            pltpu.VMEM((2, H, G), jnp.bfloat16),
            pltpu.SemaphoreType.DMA,
            pltpu.SemaphoreType.DMA,
            pltpu.SemaphoreType.DMA,
            pltpu.SemaphoreType.DMA,
            pltpu.SemaphoreType.DMA,
            pltpu.SemaphoreType.DMA,
            pltpu.SemaphoreType.DMA,
            pltpu.SemaphoreType.DMA,
        ],
        compiler_params=pltpu.CompilerParams(collective_id=0),
    )(x, dy)


# baseline (device time: 250595 ns/iter reference)
import functools

import jax
import jax.numpy as jnp
from jax import lax
from jax.experimental import pallas as pl
from jax.experimental.pallas import tpu as pltpu


def kernel(x, dy):
    K, D = x.shape
    _, F = dy.shape
    G = F // 4
    H = D // 2
    CH = 512

    def body(x_hbm, dy_hbm, out_hbm, ld, xb, dyb, pk, zs, zr, xbuf, ybuf,
             ld_sem, st_sem, zs_sem, zr_sem, xs_sem, xr_sem, ys_sem, yr_sem):
        mx = lax.axis_index("x")
        my = lax.axis_index("y")
        mz = lax.axis_index("z")
        g = 2 * mx + my
        xpeer = (1 - mx, my, mz)
        ypeer = (mx, 1 - my, mz)
        zpeer = (mx, my, 1 - mz)

        bar = pltpu.get_barrier_semaphore()
        for dev in (xpeer, ypeer, zpeer):
            pl.semaphore_signal(bar, inc=1, device_id=dev,
                                device_id_type=pl.DeviceIdType.MESH)
        pl.semaphore_wait(bar, 3)

        def load_cast(src_hbm, col0, dst):
            for c in range(G // CH):
                cp = pltpu.make_async_copy(
                    src_hbm.at[:, pl.ds(col0 + c * CH, CH)], ld, ld_sem)
                cp.start()
                cp.wait()
                dst[:, c * CH:(c + 1) * CH] = ld[...].astype(jnp.bfloat16)

        load_cast(x_hbm, 0, xb)
        load_cast(dy_hbm, g * G, dyb)

        dn = (((0,), (0,)), ((), ()))

        zs[...] = lax.dot_general(
            xb[:, pl.ds((1 - mz) * H, H)], dyb[...], dn,
            preferred_element_type=jnp.float32).astype(jnp.bfloat16)
        zrdma = pltpu.make_async_remote_copy(
            zs, zr, zs_sem, zr_sem, device_id=zpeer,
            device_id_type=pl.DeviceIdType.MESH)
        zrdma.start()

        pk[...] = lax.dot_general(
            xb[:, pl.ds(mz * H, H)], dyb[...], dn,
            preferred_element_type=jnp.float32)

        zrdma.wait()
        pk[...] = pk[...] + zr[...].astype(jnp.float32)
        xbuf[0] = pk[...].astype(jnp.bfloat16)

        st = pltpu.make_async_copy(pk, out_hbm.at[:, pl.ds(g * G, G)], st_sem)
        st.start()

        xrdma = pltpu.make_async_remote_copy(
            xbuf.at[0], xbuf.at[1], xs_sem, xr_sem, device_id=xpeer,
            device_id_type=pl.DeviceIdType.MESH)
        xrdma.start()
        xrdma.wait()

        yrdma = pltpu.make_async_remote_copy(
            xbuf, ybuf, ys_sem, yr_sem, device_id=ypeer,
            device_id_type=pl.DeviceIdType.MESH)
        yrdma.start()

        st.wait()
        pk[...] = xbuf[1].astype(jnp.float32)
        st = pltpu.make_async_copy(
            pk, out_hbm.at[:, pl.ds((2 * (1 - mx) + my) * G, G)], st_sem)
        st.start()

        yrdma.wait()
        st.wait()
        pk[...] = ybuf[0].astype(jnp.float32)
        st = pltpu.make_async_copy(
            pk, out_hbm.at[:, pl.ds((2 * mx + (1 - my)) * G, G)], st_sem)
        st.start()
        st.wait()
        pk[...] = ybuf[1].astype(jnp.float32)
        st = pltpu.make_async_copy(
            pk, out_hbm.at[:, pl.ds((2 * (1 - mx) + (1 - my)) * G, G)], st_sem)
        st.start()
        st.wait()

        @functools.partial(pl.run_scoped, sem2=pltpu.SemaphoreType.REGULAR)
        def _(sem2):
            for dev in (xpeer, ypeer, zpeer):
                pl.semaphore_signal(sem2, inc=1, device_id=dev,
                                    device_id_type=pl.DeviceIdType.MESH)
            pl.semaphore_wait(sem2, 3)

    return pl.pallas_call(
        body,
        out_shape=jax.ShapeDtypeStruct((H, F), jnp.float32),
        in_specs=[
            pl.BlockSpec(memory_space=pl.ANY),
            pl.BlockSpec(memory_space=pl.ANY),
        ],
        out_specs=pl.BlockSpec(memory_space=pl.ANY),
        scratch_shapes=[
            pltpu.VMEM((K, CH), jnp.float32),
            pltpu.VMEM((K, D), jnp.bfloat16),
            pltpu.VMEM((K, G), jnp.bfloat16),
            pltpu.VMEM((H, G), jnp.float32),
            pltpu.VMEM((H, G), jnp.bfloat16),
            pltpu.VMEM((H, G), jnp.bfloat16),
            pltpu.VMEM((2, H, G), jnp.bfloat16),
            pltpu.VMEM((2, H, G), jnp.bfloat16),
            pltpu.SemaphoreType.DMA,
            pltpu.SemaphoreType.DMA,
            pltpu.SemaphoreType.DMA,
            pltpu.SemaphoreType.DMA,
            pltpu.SemaphoreType.DMA,
            pltpu.SemaphoreType.DMA,
            pltpu.SemaphoreType.DMA,
            pltpu.SemaphoreType.DMA,
        ],
        compiler_params=pltpu.CompilerParams(
            collective_id=0,
            vmem_limit_bytes=100 * 1024 * 1024,
        ),
    )(x, dy)


# device time: 138193 ns/iter; 1.8134x vs baseline; 1.8134x over previous
import functools

import jax
import jax.numpy as jnp
from jax import lax
from jax.experimental import pallas as pl
from jax.experimental.pallas import tpu as pltpu

MESH = pl.DeviceIdType.MESH


def kernel(x, dy):
    K, D = x.shape
    _, F = dy.shape
    G = F // 4
    H = D // 2
    T = 4
    TW = G // T
    HW = TW // 2

    def body(x_hbm, dy_hbm, out_hbm, ld, xb, dyb, pk, zs, zr, sb, gx, gy,
             hx, hy, cb, ld_sem, zs_s, zr_s, axs, axr, ays, ayr,
             bxs, bxr, bys, byr, sts, stc):
        mx = lax.axis_index("x")
        my = lax.axis_index("y")
        mz = lax.axis_index("z")
        g = 2 * mx + my
        gp = 2 * (1 - mx) + my
        hh = 2 * mx + (1 - my)
        hp = 2 * (1 - mx) + (1 - my)
        xpeer = (1 - mx, my, mz)
        ypeer = (mx, 1 - my, mz)
        zpeer = (mx, my, 1 - mz)

        bar = pltpu.get_barrier_semaphore()
        for dev in (xpeer, ypeer, zpeer):
            pl.semaphore_signal(bar, inc=1, device_id=dev,
                                device_id_type=MESH)
        pl.semaphore_wait(bar, 3)

        def load_cast(src_hbm, col0, dst_ref, dst_col, slot):
            cp = pltpu.make_async_copy(
                src_hbm.at[:, pl.ds(col0, TW)], ld.at[slot], ld_sem.at[slot])
            cp.start()
            cp.wait()
            dst_ref[:, pl.ds(dst_col, TW)] = ld[slot].astype(jnp.bfloat16)

        half0 = (1 - mz) * H
        half1 = mz * H
        for c, col in enumerate((half0, half0 + TW, half1, half1 + TW)):
            load_cast(x_hbm, col, xb, col, c % 2)

        dn = (((0,), (0,)), ((), ()))

        zrd = []
        for t in range(T):
            cp = pltpu.make_async_copy(
                dy_hbm.at[:, pl.ds(g * G + t * TW, TW)], ld.at[t % 2],
                ld_sem.at[t % 2])
            cp.start()
            cp.wait()
            dyb[t % 2] = ld[t % 2].astype(jnp.bfloat16)
            zs[t] = lax.dot_general(
                xb[:, pl.ds(half0, H)], dyb[t % 2], dn,
                preferred_element_type=jnp.float32).astype(jnp.bfloat16)
            r = pltpu.make_async_remote_copy(
                zs.at[t], zr.at[t], zs_s.at[t], zr_s.at[t],
                device_id=zpeer, device_id_type=MESH)
            r.start()
            zrd.append(r)
            pk[t] = lax.dot_general(
                xb[:, pl.ds(half1, H)], dyb[t % 2], dn,
                preferred_element_type=jnp.float32)

        stcp = [None, None]
        cb_uses = [0]

        def store_via_cb(val_bf16, out_col):
            slot = cb_uses[0] % 2
            if stcp[slot] is not None:
                stcp[slot].wait()
            cb[slot] = val_bf16.astype(jnp.float32)
            cp = pltpu.make_async_copy(
                cb.at[slot], out_hbm.at[:, pl.ds(out_col, TW)], stc.at[slot])
            cp.start()
            stcp[slot] = cp
            cb_uses[0] += 1

        axd, ayd, std = [], [], []
        for t in range(T):
            zrd[t].wait()
            s = pk[t] + zr[t].astype(jnp.float32)
            pk[t] = s
            sb[2 * t] = s[:, :HW].astype(jnp.bfloat16)
            sb[2 * t + 1] = s[:, HW:].astype(jnp.bfloat16)
            ax = pltpu.make_async_remote_copy(
                sb.at[pl.ds(2 * t, 2)], gx.at[pl.ds(2 * t, 2)],
                axs.at[t], axr.at[t], device_id=xpeer, device_id_type=MESH)
            ax.start()
            axd.append(ax)
            ay = pltpu.make_async_remote_copy(
                sb.at[pl.ds(2 * t, 2)], gy.at[pl.ds(2 * t, 2)],
                ays.at[t], ayr.at[t], device_id=ypeer, device_id_type=MESH)
            ay.start()
            ayd.append(ay)
            st = pltpu.make_async_copy(
                pk.at[t], out_hbm.at[:, pl.ds(g * G + t * TW, TW)], sts.at[t])
            st.start()
            std.append(st)

        bxd, byd = [], []
        for t in range(T):
            axd[t].wait()
            by = pltpu.make_async_remote_copy(
                gx.at[2 * t + 1], hy.at[t], bys.at[t], byr.at[t],
                device_id=ypeer, device_id_type=MESH)
            by.start()
            byd.append(by)
            ayd[t].wait()
            bx = pltpu.make_async_remote_copy(
                gy.at[2 * t], hx.at[t], bxs.at[t], bxr.at[t],
                device_id=xpeer, device_id_type=MESH)
            bx.start()
            bxd.append(bx)
            store_via_cb(
                jnp.concatenate([gx[2 * t], gx[2 * t + 1]], axis=1),
                gp * G + t * TW)
            store_via_cb(
                jnp.concatenate([gy[2 * t], gy[2 * t + 1]], axis=1),
                hh * G + t * TW)

        for t in range(T):
            bxd[t].wait()
            byd[t].wait()
            store_via_cb(
                jnp.concatenate([hx[t], hy[t]], axis=1), hp * G + t * TW)

        for st in std:
            st.wait()
        for cp in stcp:
            if cp is not None:
                cp.wait()

        @functools.partial(pl.run_scoped, sem2=pltpu.SemaphoreType.REGULAR)
        def _(sem2):
            for dev in (xpeer, ypeer, zpeer):
                pl.semaphore_signal(sem2, inc=1, device_id=dev,
                                    device_id_type=MESH)
            pl.semaphore_wait(sem2, 3)

    return pl.pallas_call(
        body,
        out_shape=jax.ShapeDtypeStruct((H, F), jnp.float32),
        in_specs=[
            pl.BlockSpec(memory_space=pl.ANY),
            pl.BlockSpec(memory_space=pl.ANY),
        ],
        out_specs=pl.BlockSpec(memory_space=pl.ANY),
        scratch_shapes=[
            pltpu.VMEM((2, K, TW), jnp.float32),
            pltpu.VMEM((K, D), jnp.bfloat16),
            pltpu.VMEM((2, K, TW), jnp.bfloat16),
            pltpu.VMEM((T, H, TW), jnp.float32),
            pltpu.VMEM((T, H, TW), jnp.bfloat16),
            pltpu.VMEM((T, H, TW), jnp.bfloat16),
            pltpu.VMEM((2 * T, H, HW), jnp.bfloat16),
            pltpu.VMEM((2 * T, H, HW), jnp.bfloat16),
            pltpu.VMEM((2 * T, H, HW), jnp.bfloat16),
            pltpu.VMEM((T, H, HW), jnp.bfloat16),
            pltpu.VMEM((T, H, HW), jnp.bfloat16),
            pltpu.VMEM((2, H, TW), jnp.float32),
            pltpu.SemaphoreType.DMA((2,)),
            pltpu.SemaphoreType.DMA((T,)),
            pltpu.SemaphoreType.DMA((T,)),
            pltpu.SemaphoreType.DMA((T,)),
            pltpu.SemaphoreType.DMA((T,)),
            pltpu.SemaphoreType.DMA((T,)),
            pltpu.SemaphoreType.DMA((T,)),
            pltpu.SemaphoreType.DMA((T,)),
            pltpu.SemaphoreType.DMA((T,)),
            pltpu.SemaphoreType.DMA((T,)),
            pltpu.SemaphoreType.DMA((T,)),
            pltpu.SemaphoreType.DMA((T,)),
            pltpu.SemaphoreType.DMA((2,)),
        ],
        compiler_params=pltpu.CompilerParams(
            collective_id=0,
            vmem_limit_bytes=100 * 1024 * 1024,
        ),
    )(x, dy)
